# baseline (speedup 1.0000x reference)
import functools

import jax
import jax.numpy as jnp
from jax import lax
from jax.experimental import pallas as pl
from jax.experimental.pallas import tpu as pltpu
from jax.experimental.pallas import tpu_sc as plsc

N = 10000
E = 320000
F = 128
HD = 64
G = 64

NC = 2
NS = 16
NW = NC * NS
CHUNK = 128
NBUF = 4
NPAD = 10240
ROWS_PER_TILE = NPAD // NS
ZROWS = 64
CPT = 2560
EPAD = CPT * CHUNK
CPW_DEG = CPT // NW
CPW_C0 = 160
CPW_C1 = 0
NBUF_C0 = 4
NBUF_C1 = 4
NBUF_MAX = max(NBUF_C0, NBUF_C1)
assert NS * (CPW_C0 + CPW_C1) == CPT
assert CPW_C0 % NBUF_C0 == 0 and CPW_C1 % NBUF_C1 == 0 and CPW_DEG % NBUF == 0
assert CPW_C1 >= 0
CPW_MAX = max(CPW_C0, CPW_C1)

_MESH = plsc.VectorSubcoreMesh(core_axis_name="c", subcore_axis_name="s")
_SC_PARAMS = pltpu.CompilerParams(use_tc_tiling_on_sc=False)


def _zero_fill(zbuf, width):
    z16 = jnp.zeros((16,), jnp.float32)

    def row(i, _):
        for k in range(width // 16):
            zbuf[i, pl.ds(k * 16, 16)] = z16
        return 0

    lax.fori_loop(0, zbuf.shape[0], row, 0)


def _zero_acc_slice(zbuf, acc, s):
    zr = zbuf.shape[0]

    def cp(k, _):
        pltpu.sync_copy(zbuf, acc.at[pl.ds(s * ROWS_PER_TILE + k * zr, zr)])
        return 0

    lax.fori_loop(0, ROWS_PER_TILE // zr, cp, 0)


def _deg_body(dst_hbm, out_hbm, dst_v, ones_v, zbuf, acc, sem):
    c = lax.axis_index("c")
    s = lax.axis_index("s")
    wid = c * NS + s
    _zero_fill(zbuf, 16)
    one16 = jnp.ones((16,), jnp.float32)

    def fill_ones(i, _):
        ones_v[i, :] = one16
        return 0

    lax.fori_loop(0, CHUNK, fill_ones, 0)
    pltpu.sync_copy(dst_hbm.at[pl.ds(wid * CPW_DEG, CPW_DEG)], dst_v)
    _zero_acc_slice(zbuf, acc, s)
    plsc.subcore_barrier()

    for b in range(NBUF):
        pltpu.async_copy(ones_v, acc.at[dst_v.at[b]], sem, add=True)

    def body(j, _):
        pltpu.make_async_copy(ones_v, acc.at[dst_v.at[j]], sem).wait()

        @pl.when(j + NBUF < CPW_DEG)
        def _():
            pltpu.async_copy(ones_v, acc.at[dst_v.at[j + NBUF]], sem, add=True)

        return 0

    lax.fori_loop(0, CPW_DEG, body, 0)
    plsc.subcore_barrier()
    pltpu.sync_copy(acc.at[pl.ds(s * ROWS_PER_TILE, ROWS_PER_TILE)],
                    out_hbm.at[c, pl.ds(s * ROWS_PER_TILE, ROWS_PER_TILE)])


_deg_kernel = functools.partial(
    pl.kernel,
    out_type=jax.ShapeDtypeStruct((NC, NPAD, 16), jnp.float32),
    mesh=_MESH,
    compiler_params=_SC_PARAMS,
    scratch_types=[
        pltpu.VMEM((CPW_DEG, CHUNK), jnp.int32),
        pltpu.VMEM((CHUNK, 16), jnp.float32),
        pltpu.VMEM((ZROWS, 16), jnp.float32),
        pltpu.VMEM_SHARED((NPAD, 16), jnp.float32),
        pltpu.SemaphoreType.DMA,
    ],
)(_deg_body)


def _scat_body(g_hbm, src_hbm, dst_hbm, out_hbm, src_v, dst_v, rows_v, zbuf, acc,
               gsem):
    c = lax.axis_index("c")
    s = lax.axis_index("s")
    _zero_fill(zbuf, HD)
    _zero_acc_slice(zbuf, acc, s)

    def pipeline(chunk0, n, nbuf):
        pltpu.sync_copy(src_hbm.at[pl.ds(chunk0, n)], src_v.at[pl.ds(0, n)])
        pltpu.sync_copy(dst_hbm.at[pl.ds(chunk0, n)], dst_v.at[pl.ds(0, n)])
        plsc.subcore_barrier()

        for b in range(nbuf):
            pltpu.async_copy(g_hbm.at[src_v.at[b]], rows_v.at[b], gsem.at[b])

        def outer(jo, _):
            for b in range(nbuf):
                j = jo * nbuf + b
                pltpu.make_async_copy(g_hbm.at[src_v.at[j]], rows_v.at[b],
                                      gsem.at[b]).wait()
                pltpu.sync_copy(rows_v.at[b], acc.at[dst_v.at[j]], add=True)

                @pl.when(j + nbuf < n)
                def _():
                    pltpu.async_copy(g_hbm.at[src_v.at[j + nbuf]], rows_v.at[b],
                                     gsem.at[b])

            return 0

        lax.fori_loop(0, n // nbuf, outer, 0)

    @pl.when(c == 0)
    def _():
        pipeline(s * CPW_C0, CPW_C0, NBUF_C0)

    if CPW_C1 > 0:
        @pl.when(c == 1)
        def _():
            pipeline(NS * CPW_C0 + s * CPW_C1, CPW_C1, NBUF_C1)

    plsc.subcore_barrier()
    pltpu.sync_copy(acc.at[pl.ds(s * ROWS_PER_TILE, ROWS_PER_TILE)],
                    out_hbm.at[c, pl.ds(s * ROWS_PER_TILE, ROWS_PER_TILE)])


_scat_kernel = functools.partial(
    pl.kernel,
    out_type=jax.ShapeDtypeStruct((NC, NPAD, HD), jnp.float32),
    mesh=_MESH,
    compiler_params=_SC_PARAMS,
    scratch_types=[
        pltpu.VMEM((CPW_MAX, CHUNK), jnp.int32),
        pltpu.VMEM((CPW_MAX, CHUNK), jnp.int32),
        pltpu.VMEM((NBUF_MAX, CHUNK, HD), jnp.float32),
        pltpu.VMEM((ZROWS // 2, HD), jnp.float32),
        pltpu.VMEM_SHARED((NPAD, HD), jnp.float32),
        pltpu.SemaphoreType.DMA((NBUF_MAX,)),
    ],
)(_scat_body)


def _dinv(degp_ref):
    deg = degp_ref[0, :, 0:1] + degp_ref[1, :, 0:1] + 1.0
    return lax.rsqrt(deg)


def _mm1_body(x_ref, w1_ref, degp_ref, g1_ref):
    dinv = _dinv(degp_ref)
    g1_ref[...] = jnp.dot(x_ref[...], w1_ref[...],
                          preferred_element_type=jnp.float32) * dinv


def _mm2_body(s1_ref, g1_ref, degp_ref, b1_ref, w2_ref, g2_ref):
    dinv = _dinv(degp_ref)
    h1 = jnp.maximum((s1_ref[0] + s1_ref[1] + g1_ref[...]) * dinv + b1_ref[...], 0.0)
    g2_ref[...] = jnp.dot(h1, w2_ref[...],
                          preferred_element_type=jnp.float32) * dinv


def _head_body(s2_ref, g2_ref, degp_ref, b2_ref, batch_ref, wfc_ref, bfc_ref, out_ref):
    dinv = _dinv(degp_ref)
    h2 = jnp.maximum((s2_ref[0] + s2_ref[1] + g2_ref[...]) * dinv + b2_ref[...], 0.0)
    onehot = (batch_ref[...] == lax.broadcasted_iota(jnp.int32, (G, NPAD), 0)
              ).astype(jnp.float32)
    sums = jnp.dot(onehot, h2, preferred_element_type=jnp.float32)
    cnt = jnp.dot(onehot, jnp.ones((NPAD, 1), jnp.float32),
                  preferred_element_type=jnp.float32)
    pooled = sums / jnp.maximum(cnt, 1.0)
    z = jnp.dot(pooled, wfc_ref[...], preferred_element_type=jnp.float32) + bfc_ref[...]
    out_ref[...] = 1.0 / (1.0 + jnp.exp(-z))


def kernel(x, edge_index, batch, W1, b1, W2, b2, Wfc, bfc):
    src = jnp.full((EPAD,), N, jnp.int32).at[:E].set(
        edge_index[0].astype(jnp.int32)).reshape(CPT, CHUNK)
    dst = jnp.full((EPAD,), N, jnp.int32).at[:E].set(
        edge_index[1].astype(jnp.int32)).reshape(CPT, CHUNK)
    x_pad = jnp.zeros((NPAD, F), jnp.float32).at[:N].set(x)
    batch_pad = jnp.full((1, NPAD), G, jnp.int32).at[0, :N].set(batch.astype(jnp.int32))
    b1r = b1.reshape(1, HD)
    b2r = b2.reshape(1, HD)
    bfcr = bfc.reshape(1, 1)

    degp = _deg_kernel(dst)

    g1 = pl.pallas_call(
        _mm1_body,
        out_shape=jax.ShapeDtypeStruct((NPAD, HD), jnp.float32),
    )(x_pad, W1, degp)

    s1 = _scat_kernel(g1, src, dst)

    g2 = pl.pallas_call(
        _mm2_body,
        out_shape=jax.ShapeDtypeStruct((NPAD, HD), jnp.float32),
    )(s1, g1, degp, b1r, W2)

    s2 = _scat_kernel(g2, src, dst)

    out = pl.pallas_call(
        _head_body,
        out_shape=jax.ShapeDtypeStruct((G, 1), jnp.float32),
    )(s2, g2, degp, b2r, batch_pad, Wfc, bfcr)
    return out


# speedup vs baseline: 2.7913x; 2.7913x over previous
import functools

import jax
import jax.numpy as jnp
from jax import lax
from jax.experimental import pallas as pl
from jax.experimental.pallas import tpu as pltpu
from jax.experimental.pallas import tpu_sc as plsc

N = 10000
E = 320000
F = 128
HD = 64
G = 64

NC = 2
NS = 16
NW = NC * NS
CHUNK = 128
NBUF = 4
NPAD = 10240
ROWS_PER_TILE = NPAD // NS
ZROWS = 64
CPT = 2560
EPAD = CPT * CHUNK
CPW_DEG = CPT // NW
CPW_SC = CPT // NW
NBUF_SC = 8
assert CPW_SC % NBUF_SC == 0 and CPW_DEG % NBUF == 0

_MESH = plsc.VectorSubcoreMesh(core_axis_name="c", subcore_axis_name="s")
_SC_PARAMS = pltpu.CompilerParams(use_tc_tiling_on_sc=False)


def _zero_fill(zbuf, width):
    z16 = jnp.zeros((16,), jnp.float32)

    def row(i, _):
        for k in range(width // 16):
            zbuf[i, pl.ds(k * 16, 16)] = z16
        return 0

    lax.fori_loop(0, zbuf.shape[0], row, 0)


def _zero_acc_slice(zbuf, acc, s):
    zr = zbuf.shape[0]

    def cp(k, _):
        pltpu.sync_copy(zbuf, acc.at[pl.ds(s * ROWS_PER_TILE + k * zr, zr)])
        return 0

    lax.fori_loop(0, ROWS_PER_TILE // zr, cp, 0)


def _deg_body(dst_hbm, out_hbm, dst_v, ones_v, zbuf, acc, sem):
    c = lax.axis_index("c")
    s = lax.axis_index("s")
    wid = c * NS + s
    _zero_fill(zbuf, 16)
    one16 = jnp.ones((16,), jnp.float32)

    def fill_ones(i, _):
        ones_v[i, :] = one16
        return 0

    lax.fori_loop(0, CHUNK, fill_ones, 0)
    pltpu.sync_copy(dst_hbm.at[pl.ds(wid * CPW_DEG, CPW_DEG)], dst_v)
    _zero_acc_slice(zbuf, acc, s)
    plsc.subcore_barrier()

    for b in range(NBUF):
        pltpu.async_copy(ones_v, acc.at[dst_v.at[b]], sem, add=True)

    def body(j, _):
        pltpu.make_async_copy(ones_v, acc.at[dst_v.at[j]], sem).wait()

        @pl.when(j + NBUF < CPW_DEG)
        def _():
            pltpu.async_copy(ones_v, acc.at[dst_v.at[j + NBUF]], sem, add=True)

        return 0

    lax.fori_loop(0, CPW_DEG, body, 0)
    plsc.subcore_barrier()
    pltpu.sync_copy(acc.at[pl.ds(s * ROWS_PER_TILE, ROWS_PER_TILE)],
                    out_hbm.at[c, pl.ds(s * ROWS_PER_TILE, ROWS_PER_TILE)])


_deg_kernel = functools.partial(
    pl.kernel,
    out_type=jax.ShapeDtypeStruct((NC, NPAD, 16), jnp.float32),
    mesh=_MESH,
    compiler_params=_SC_PARAMS,
    scratch_types=[
        pltpu.VMEM((CPW_DEG, CHUNK), jnp.int32),
        pltpu.VMEM((CHUNK, 16), jnp.float32),
        pltpu.VMEM((ZROWS, 16), jnp.float32),
        pltpu.VMEM_SHARED((NPAD, 16), jnp.float32),
        pltpu.SemaphoreType.DMA,
    ],
)(_deg_body)


def _scat_body(g_hbm, src_hbm, dst_hbm, out_hbm, src_v, dst_v, rows_v, zbuf, acc,
               gsem):
    c = lax.axis_index("c")
    s = lax.axis_index("s")
    wid = c * NS + s
    _zero_fill(zbuf, HD)
    _zero_acc_slice(zbuf, acc, s)
    chunk0 = wid * CPW_SC
    pltpu.sync_copy(src_hbm.at[pl.ds(chunk0, CPW_SC)], src_v)
    pltpu.sync_copy(dst_hbm.at[pl.ds(chunk0, CPW_SC)], dst_v)
    plsc.subcore_barrier()

    for b in range(NBUF_SC):
        pltpu.async_copy(g_hbm.at[src_v.at[b]], rows_v.at[b], gsem.at[b])

    def outer(jo, _):
        for b in range(NBUF_SC):
            j = jo * NBUF_SC + b
            pltpu.make_async_copy(g_hbm.at[src_v.at[j]], rows_v.at[b],
                                  gsem.at[b]).wait()
            pltpu.sync_copy(rows_v.at[b], acc.at[dst_v.at[j]], add=True)

            @pl.when(j + NBUF_SC < CPW_SC)
            def _():
                pltpu.async_copy(g_hbm.at[src_v.at[j + NBUF_SC]], rows_v.at[b],
                                 gsem.at[b])

        return 0

    lax.fori_loop(0, CPW_SC // NBUF_SC, outer, 0)
    plsc.subcore_barrier()
    pltpu.sync_copy(acc.at[pl.ds(s * ROWS_PER_TILE, ROWS_PER_TILE)],
                    out_hbm.at[c, pl.ds(s * ROWS_PER_TILE, ROWS_PER_TILE)])


_scat_kernel = functools.partial(
    pl.kernel,
    out_type=jax.ShapeDtypeStruct((NC, NPAD, HD), jnp.float32),
    mesh=_MESH,
    compiler_params=_SC_PARAMS,
    scratch_types=[
        pltpu.VMEM((CPW_SC, CHUNK), jnp.int32),
        pltpu.VMEM((CPW_SC, CHUNK), jnp.int32),
        pltpu.VMEM((NBUF_SC, CHUNK, HD), jnp.float32),
        pltpu.VMEM((ZROWS // 2, HD), jnp.float32),
        pltpu.VMEM_SHARED((NPAD, HD), jnp.float32),
        pltpu.SemaphoreType.DMA((NBUF_SC,)),
    ],
)(_scat_body)


def _dinv(degp_ref):
    deg = degp_ref[0, :, 0:1] + degp_ref[1, :, 0:1] + 1.0
    return lax.rsqrt(deg)


def _mm1_body(x_ref, w1_ref, degp_ref, g1_ref):
    dinv = _dinv(degp_ref)
    g1_ref[...] = jnp.dot(x_ref[...], w1_ref[...],
                          preferred_element_type=jnp.float32) * dinv


def _mm2_body(s1_ref, g1_ref, degp_ref, b1_ref, w2_ref, g2_ref):
    dinv = _dinv(degp_ref)
    h1 = jnp.maximum((s1_ref[0] + s1_ref[1] + g1_ref[...]) * dinv + b1_ref[...], 0.0)
    g2_ref[...] = jnp.dot(h1, w2_ref[...],
                          preferred_element_type=jnp.float32) * dinv


def _head_body(s2_ref, g2_ref, degp_ref, b2_ref, batch_ref, wfc_ref, bfc_ref, out_ref):
    dinv = _dinv(degp_ref)
    h2 = jnp.maximum((s2_ref[0] + s2_ref[1] + g2_ref[...]) * dinv + b2_ref[...], 0.0)
    onehot = (batch_ref[...] == lax.broadcasted_iota(jnp.int32, (G, NPAD), 0)
              ).astype(jnp.float32)
    sums = jnp.dot(onehot, h2, preferred_element_type=jnp.float32)
    cnt = jnp.dot(onehot, jnp.ones((NPAD, 1), jnp.float32),
                  preferred_element_type=jnp.float32)
    pooled = sums / jnp.maximum(cnt, 1.0)
    z = jnp.dot(pooled, wfc_ref[...], preferred_element_type=jnp.float32) + bfc_ref[...]
    out_ref[...] = 1.0 / (1.0 + jnp.exp(-z))


def kernel(x, edge_index, batch, W1, b1, W2, b2, Wfc, bfc):
    pad_idx = (jnp.arange(EPAD - E, dtype=jnp.int32) % (NPAD - N)) + N
    src = jnp.concatenate(
        [edge_index[0].astype(jnp.int32), pad_idx]).reshape(CPT, CHUNK)
    dst = jnp.concatenate(
        [edge_index[1].astype(jnp.int32), pad_idx]).reshape(CPT, CHUNK)
    x_pad = jnp.zeros((NPAD, F), jnp.float32).at[:N].set(x)
    batch_pad = jnp.full((1, NPAD), G, jnp.int32).at[0, :N].set(batch.astype(jnp.int32))
    b1r = b1.reshape(1, HD)
    b2r = b2.reshape(1, HD)
    bfcr = bfc.reshape(1, 1)

    degp = _deg_kernel(dst)

    g1 = pl.pallas_call(
        _mm1_body,
        out_shape=jax.ShapeDtypeStruct((NPAD, HD), jnp.float32),
    )(x_pad, W1, degp)

    s1 = _scat_kernel(g1, src, dst)

    g2 = pl.pallas_call(
        _mm2_body,
        out_shape=jax.ShapeDtypeStruct((NPAD, HD), jnp.float32),
    )(s1, g1, degp, b1r, W2)

    s2 = _scat_kernel(g2, src, dst)

    out = pl.pallas_call(
        _head_body,
        out_shape=jax.ShapeDtypeStruct((G, 1), jnp.float32),
    )(s2, g2, degp, b2r, batch_pad, Wfc, bfcr)
    return out


# speedup vs baseline: 2.7949x; 1.0013x over previous
import functools

import jax
import jax.numpy as jnp
from jax import lax
from jax.experimental import pallas as pl
from jax.experimental.pallas import tpu as pltpu
from jax.experimental.pallas import tpu_sc as plsc

N = 10000
E = 320000
F = 128
HD = 64
G = 64

NC = 2
NS = 16
NW = NC * NS
CHUNK = 128
NBUF = 4
NPAD = 10240
ROWS_PER_TILE = NPAD // NS
ZROWS = 64
CPT = 2560
EPAD = CPT * CHUNK
CPW_DEG = CPT // NW
CPW_SC = CPT // NW
NBUF_SC = 8
assert CPW_SC % NBUF_SC == 0 and CPW_DEG % NBUF == 0

_MESH = plsc.VectorSubcoreMesh(core_axis_name="c", subcore_axis_name="s")
_SC_PARAMS = pltpu.CompilerParams(use_tc_tiling_on_sc=False)


def _zero_fill(zbuf, width):
    z16 = jnp.zeros((16,), jnp.float32)

    def row(i, _):
        for k in range(width // 16):
            zbuf[i, pl.ds(k * 16, 16)] = z16
        return 0

    lax.fori_loop(0, zbuf.shape[0], row, 0)


def _zero_acc_slice(zbuf, acc, s):
    zr = zbuf.shape[0]

    def cp(k, _):
        pltpu.sync_copy(zbuf, acc.at[pl.ds(s * ROWS_PER_TILE + k * zr, zr)])
        return 0

    lax.fori_loop(0, ROWS_PER_TILE // zr, cp, 0)


def _deg_body(dst_hbm, out_hbm, dst_v, ones_v, zbuf, acc, sem):
    c = lax.axis_index("c")
    s = lax.axis_index("s")
    wid = c * NS + s
    _zero_fill(zbuf, 16)
    one16 = jnp.ones((16,), jnp.float32)

    def fill_ones(i, _):
        ones_v[i, :] = one16
        return 0

    lax.fori_loop(0, CHUNK, fill_ones, 0)
    pltpu.sync_copy(dst_hbm.at[pl.ds(wid * CPW_DEG, CPW_DEG)], dst_v)
    _zero_acc_slice(zbuf, acc, s)
    plsc.subcore_barrier()

    for b in range(NBUF):
        pltpu.async_copy(ones_v, acc.at[dst_v.at[b]], sem, add=True)

    def body(j, _):
        pltpu.make_async_copy(ones_v, acc.at[dst_v.at[j]], sem).wait()

        @pl.when(j + NBUF < CPW_DEG)
        def _():
            pltpu.async_copy(ones_v, acc.at[dst_v.at[j + NBUF]], sem, add=True)

        return 0

    lax.fori_loop(0, CPW_DEG, body, 0)
    plsc.subcore_barrier()
    pltpu.sync_copy(acc.at[pl.ds(s * ROWS_PER_TILE, ROWS_PER_TILE)],
                    out_hbm.at[c, pl.ds(s * ROWS_PER_TILE, ROWS_PER_TILE)])


_deg_kernel = functools.partial(
    pl.kernel,
    out_type=jax.ShapeDtypeStruct((NC, NPAD, 16), jnp.float32),
    mesh=_MESH,
    compiler_params=_SC_PARAMS,
    scratch_types=[
        pltpu.VMEM((CPW_DEG, CHUNK), jnp.int32),
        pltpu.VMEM((CHUNK, 16), jnp.float32),
        pltpu.VMEM((ZROWS, 16), jnp.float32),
        pltpu.VMEM_SHARED((NPAD, 16), jnp.float32),
        pltpu.SemaphoreType.DMA,
    ],
)(_deg_body)


def _scat_body(g_hbm, src_hbm, dst_hbm, out_hbm, src_v, dst_v, rows_v, zbuf, acc,
               gsem):
    c = lax.axis_index("c")
    s = lax.axis_index("s")
    wid = c * NS + s
    _zero_fill(zbuf, HD)
    _zero_acc_slice(zbuf, acc, s)
    chunk0 = wid * CPW_SC
    pltpu.sync_copy(src_hbm.at[pl.ds(chunk0, CPW_SC)], src_v)
    pltpu.sync_copy(dst_hbm.at[pl.ds(chunk0, CPW_SC)], dst_v)
    plsc.subcore_barrier()

    for b in range(NBUF_SC):
        pltpu.async_copy(g_hbm.at[src_v.at[b]], rows_v.at[b], gsem.at[b])

    def outer(jo, _):
        for b in range(NBUF_SC):
            j = jo * NBUF_SC + b
            pltpu.make_async_copy(g_hbm.at[src_v.at[j]], rows_v.at[b],
                                  gsem.at[b]).wait()
            pltpu.sync_copy(rows_v.at[b], acc.at[dst_v.at[j]], add=True)

            @pl.when(j + NBUF_SC < CPW_SC)
            def _():
                pltpu.async_copy(g_hbm.at[src_v.at[j + NBUF_SC]], rows_v.at[b],
                                 gsem.at[b])

        return 0

    lax.fori_loop(0, CPW_SC // NBUF_SC, outer, 0)
    plsc.subcore_barrier()
    pltpu.sync_copy(acc.at[pl.ds(s * ROWS_PER_TILE, ROWS_PER_TILE)],
                    out_hbm.at[c, pl.ds(s * ROWS_PER_TILE, ROWS_PER_TILE)])


_scat_kernel = functools.partial(
    pl.kernel,
    out_type=jax.ShapeDtypeStruct((NC, NPAD, HD), jnp.float32),
    mesh=_MESH,
    compiler_params=_SC_PARAMS,
    scratch_types=[
        pltpu.VMEM((CPW_SC, CHUNK), jnp.int32),
        pltpu.VMEM((CPW_SC, CHUNK), jnp.int32),
        pltpu.VMEM((NBUF_SC, CHUNK, HD), jnp.float32),
        pltpu.VMEM((ZROWS // 2, HD), jnp.float32),
        pltpu.VMEM_SHARED((NPAD, HD), jnp.float32),
        pltpu.SemaphoreType.DMA((NBUF_SC,)),
    ],
)(_scat_body)


def _dinv(degp_ref):
    deg = degp_ref[0, :, 0:1] + degp_ref[1, :, 0:1] + 1.0
    return lax.rsqrt(deg)


def _mm1raw_body(x_ref, w1_ref, h_ref):
    h_ref[...] = jnp.dot(x_ref[...], w1_ref[...],
                         preferred_element_type=jnp.float32)


def _scale_body(h_ref, degp_ref, g1_ref):
    g1_ref[...] = h_ref[...] * _dinv(degp_ref)


def _mm2_body(s1_ref, g1_ref, degp_ref, b1_ref, w2_ref, g2_ref):
    dinv = _dinv(degp_ref)
    h1 = jnp.maximum((s1_ref[0] + s1_ref[1] + g1_ref[...]) * dinv + b1_ref[...], 0.0)
    g2_ref[...] = jnp.dot(h1, w2_ref[...],
                          preferred_element_type=jnp.float32) * dinv


def _head_body(s2_ref, g2_ref, degp_ref, b2_ref, batch_ref, wfc_ref, bfc_ref, out_ref):
    dinv = _dinv(degp_ref)
    h2 = jnp.maximum((s2_ref[0] + s2_ref[1] + g2_ref[...]) * dinv + b2_ref[...], 0.0)
    onehot = (batch_ref[...] == lax.broadcasted_iota(jnp.int32, (G, NPAD), 0)
              ).astype(jnp.float32)
    sums = jnp.dot(onehot, h2, preferred_element_type=jnp.float32)
    cnt = jnp.dot(onehot, jnp.ones((NPAD, 1), jnp.float32),
                  preferred_element_type=jnp.float32)
    pooled = sums / jnp.maximum(cnt, 1.0)
    z = jnp.dot(pooled, wfc_ref[...], preferred_element_type=jnp.float32) + bfc_ref[...]
    out_ref[...] = 1.0 / (1.0 + jnp.exp(-z))


def kernel(x, edge_index, batch, W1, b1, W2, b2, Wfc, bfc):
    pad_idx = (jnp.arange(EPAD - E, dtype=jnp.int32) % (NPAD - N)) + N
    src = jnp.concatenate(
        [edge_index[0].astype(jnp.int32), pad_idx]).reshape(CPT, CHUNK)
    dst = jnp.concatenate(
        [edge_index[1].astype(jnp.int32), pad_idx]).reshape(CPT, CHUNK)
    x_pad = jnp.zeros((NPAD, F), jnp.float32).at[:N].set(x)
    batch_pad = jnp.full((1, NPAD), G, jnp.int32).at[0, :N].set(batch.astype(jnp.int32))
    b1r = b1.reshape(1, HD)
    b2r = b2.reshape(1, HD)
    bfcr = bfc.reshape(1, 1)

    degp = _deg_kernel(dst)

    h1raw = pl.pallas_call(
        _mm1raw_body,
        out_shape=jax.ShapeDtypeStruct((NPAD, HD), jnp.float32),
    )(x_pad, W1)

    g1 = pl.pallas_call(
        _scale_body,
        out_shape=jax.ShapeDtypeStruct((NPAD, HD), jnp.float32),
    )(h1raw, degp)

    s1 = _scat_kernel(g1, src, dst)

    g2 = pl.pallas_call(
        _mm2_body,
        out_shape=jax.ShapeDtypeStruct((NPAD, HD), jnp.float32),
    )(s1, g1, degp, b1r, W2)

    s2 = _scat_kernel(g2, src, dst)

    out = pl.pallas_call(
        _head_body,
        out_shape=jax.ShapeDtypeStruct((G, 1), jnp.float32),
    )(s2, g2, degp, b2r, batch_pad, Wfc, bfcr)
    return out
